# Initial kernel scaffold; baseline (speedup 1.0000x reference)
import jax
import jax.numpy as jnp
from jax.experimental import pallas as pl


def kernel(coordinates, features, neighbour_indices):
    raise NotImplementedError("write your pallas kernel here")


# speedup vs baseline: 28.6166x; 28.6166x over previous
import functools

import jax
import jax.numpy as jnp
from jax import lax
from jax.experimental import pallas as pl
from jax.experimental.pallas import tpu as pltpu
from jax.experimental.pallas import tpu_sc as plsc

_L = 16
_NC, _NS = 2, 16
_NW = _NC * _NS


def _knn_mean_kernel(Vp, ndim, nfeat, K, C, scale):
    assert K % _L == 0 and nfeat % _L == 0 and C % 2 == 0
    nh = K // _L
    nj = nfeat // _L
    mesh = plsc.VectorSubcoreMesh(
        core_axis_name="c", subcore_axis_name="s",
        num_cores=_NC, num_subcores=_NS)

    @functools.partial(
        pl.kernel,
        mesh=mesh,
        compiler_params=pltpu.CompilerParams(needs_layout_passes=False),
        out_type=jax.ShapeDtypeStruct((Vp * nfeat,), jnp.float32),
        scratch_types=[
            pltpu.VMEM((Vp * ndim,), jnp.float32),
            pltpu.VMEM((C * K,), jnp.int32),
            pltpu.VMEM((K, nfeat), jnp.float32),
            pltpu.VMEM((K, nfeat), jnp.float32),
            pltpu.VMEM((C * nfeat,), jnp.float32),
            pltpu.SemaphoreType.DMA,
            pltpu.SemaphoreType.DMA,
        ],
    )
    def knn(coords_hbm, feat_hbm, idx_hbm, out_hbm,
            cal, idxv, fb0, fb1, obuf, s0, s1):
        wid = lax.axis_index("s") * _NC + lax.axis_index("c")
        base = wid * C
        pltpu.sync_copy(coords_hbm, cal)
        pltpu.sync_copy(idx_hbm.at[pl.ds(base * K, C * K)], idxv)

        def fire(i, fb, sem):
            pltpu.async_copy(feat_hbm.at[idxv.at[pl.ds(i * K, K)]], fb, sem)

        def wait(fb, sem):
            pltpu.make_async_copy(
                feat_hbm.at[idxv.at[pl.ds(0, K)]], fb, sem).wait()

        def weights(i):
            gi = base + i
            ci = [
                plsc.load_gather(
                    cal, [jnp.full((_L,), gi * ndim + d, jnp.int32)])
                for d in range(ndim)
            ]
            w = []
            for h in range(nh):
                nidx = idxv[pl.ds(i * K + h * _L, _L)]
                na = nidx * ndim
                dist = jnp.zeros((_L,), jnp.float32)
                for d in range(ndim):
                    cn = plsc.load_gather(
                        cal, [na + jnp.full((_L,), d, jnp.int32)])
                    df = cn - ci[d]
                    dist = dist + df * df
                w.append(jnp.exp(dist * (-scale)))
            return w

        def accum(i, w, fb):
            acc = [jnp.zeros((_L,), jnp.float32) for _ in range(nj)]
            for k in range(K):
                wk = jnp.take_along_axis(
                    w[k // _L], jnp.full((_L,), k % _L, jnp.int32), axis=0)
                for j in range(nj):
                    acc[j] = acc[j] + wk * fb[k, pl.ds(j * _L, _L)]
            inv = 1.0 / K
            for j in range(nj):
                obuf[pl.ds(i * nfeat + j * _L, _L)] = acc[j] * inv

        fire(0, fb0, s0)

        def body(t, carry):
            i = t * 2
            fire(i + 1, fb1, s1)
            w0 = weights(i)
            wait(fb0, s0)
            accum(i, w0, fb0)

            @pl.when(t < C // 2 - 1)
            def _():
                fire(i + 2, fb0, s0)

            w1 = weights(i + 1)
            wait(fb1, s1)
            accum(i + 1, w1, fb1)
            return carry

        lax.fori_loop(0, C // 2, body, 0)
        pltpu.sync_copy(obuf, out_hbm.at[pl.ds(base * nfeat, C * nfeat)])

    return knn


def kernel(coordinates, features, neighbour_indices):
    V, ndim = coordinates.shape
    nfeat = features.shape[1]
    K = neighbour_indices.shape[1]
    C = -(-V // (2 * _NW)) * 2
    Vp = C * _NW
    coords_pad = (
        jnp.zeros((Vp, ndim), jnp.float32)
        .at[:V].set(coordinates)
        .reshape(Vp * ndim)
    )
    idx_pad = (
        jnp.zeros((Vp, K), jnp.int32)
        .at[:V].set(neighbour_indices)
        .reshape(Vp * K)
    )
    knn = _knn_mean_kernel(Vp, ndim, nfeat, K, C, scale=10.0)
    f = knn(coords_pad, features, idx_pad).reshape(Vp, nfeat)[:V]
    return jnp.concatenate([f] * (2 * ndim + 1), axis=1)
